# baseline (speedup 1.0000x reference)
import functools

import jax
import jax.numpy as jnp
from jax import lax
from jax.experimental import pallas as pl
from jax.experimental.pallas import tpu as pltpu
from jax.experimental.pallas import tpu_sc as plsc

_NC = 2
_NS = 16
_NW = _NC * _NS
_CH = 128


def _sc_mesh():
  return plsc.VectorSubcoreMesh(
      core_axis_name="c", subcore_axis_name="s", num_cores=_NC,
      num_subcores=_NS)


@functools.lru_cache(maxsize=None)
def _make_gather(n_nodes, n_edges, feat, dtype):
  n_chunks = n_edges // _CH
  per_w = -(-n_chunks // _NW)

  @functools.partial(
      pl.kernel,
      out_type=(
          jax.ShapeDtypeStruct((n_edges, feat), dtype),
          jax.ShapeDtypeStruct((n_edges, feat), dtype),
      ),
      mesh=_sc_mesh(),
      scratch_types=[
          pltpu.VMEM((_CH,), jnp.int32),
          pltpu.VMEM((_CH, feat), dtype),
          pltpu.SemaphoreType.DMA,
      ],
      compiler_params=pltpu.CompilerParams(use_tc_tiling_on_sc=False),
  )
  def gather_kernel(h_hbm, src_hbm, dst_hbm, osrc_hbm, odst_hbm,
                    idx_v, rows_v, sem):
    wid = lax.axis_index("s") * _NC + lax.axis_index("c")
    for idx_hbm, out_hbm in ((src_hbm, osrc_hbm), (dst_hbm, odst_hbm)):
      @pl.loop(0, per_w)
      def _(c, idx_hbm=idx_hbm, out_hbm=out_hbm):
        ch = wid + _NW * c

        @pl.when(ch < n_chunks)
        def _():
          base = ch * _CH
          pltpu.sync_copy(idx_hbm.at[pl.ds(base, _CH)], idx_v)
          pltpu.async_copy(h_hbm.at[idx_v], rows_v, sem).wait()
          pltpu.sync_copy(rows_v, out_hbm.at[pl.ds(base, _CH)])

  return gather_kernel


@functools.lru_cache(maxsize=None)
def _make_scatter(n_nodes, n_edges, feat):
  n_chunks = n_edges // _CH
  per_w = -(-n_chunks // _NW)
  out_ch = 80
  n_out_chunks = -(-n_nodes // out_ch)
  out_per_sub = -(-n_out_chunks // _NS)

  @functools.partial(
      pl.kernel,
      out_type=jax.ShapeDtypeStruct((_NC, n_nodes, feat), jnp.float32),
      mesh=_sc_mesh(),
      scratch_types=[
          pltpu.VMEM((_CH,), jnp.int32),
          pltpu.VMEM((_CH, feat), jnp.float32),
          pltpu.VMEM_SHARED((n_nodes, feat), jnp.float32),
      ],
  )
  def scatter_kernel(rows_hbm, dst_hbm, zero_hbm, out_hbm,
                     idx_v, rows_v, acc_sh):
    cid = lax.axis_index("c")
    sid = lax.axis_index("s")
    wid = sid * _NC + cid

    @pl.when(sid == 0)
    def _():
      pltpu.sync_copy(zero_hbm, acc_sh)

    plsc.subcore_barrier()

    @pl.loop(0, per_w)
    def _(c):
      ch = wid + _NW * c

      @pl.when(ch < n_chunks)
      def _():
        base = ch * _CH
        pltpu.sync_copy(dst_hbm.at[pl.ds(base, _CH)], idx_v)
        pltpu.sync_copy(rows_hbm.at[pl.ds(base, _CH)], rows_v)
        pltpu.sync_copy(rows_v, acc_sh.at[idx_v], add=True)

    plsc.subcore_barrier()

    @pl.loop(0, out_per_sub)
    def _(c):
      ch = sid + _NS * c

      @pl.when(ch < n_out_chunks)
      def _():
        base = ch * out_ch
        pltpu.sync_copy(acc_sh.at[pl.ds(base, out_ch)],
                        out_hbm.at[cid, pl.ds(base, out_ch)])

  return scatter_kernel


def _mlp_body(n_parts, has_res, out_bf16, *refs):
  xs = refs[:n_parts]
  w1s = refs[n_parts:2 * n_parts]
  w2, w3, w4 = refs[2 * n_parts:2 * n_parts + 3]
  b1, b2, b3, b4 = refs[2 * n_parts + 3:2 * n_parts + 7]
  pos = 2 * n_parts + 7
  res = refs[pos] if has_res else None
  pos += 1 if has_res else 0
  out = refs[pos]
  out_b = refs[pos + 1] if out_bf16 else None

  def dot1(x, w):
    return jnp.dot(x, w.astype(x.dtype), preferred_element_type=jnp.float32)

  z = dot1(xs[0][...], w1s[0][...])
  for j in range(1, n_parts):
    z += dot1(xs[j][...], w1s[j][...])
  z = jnp.maximum(z + b1[...], 0.0)
  z = jnp.maximum(
      jnp.dot(z, w2[...], preferred_element_type=jnp.float32) + b2[...], 0.0)
  z = jnp.maximum(
      jnp.dot(z, w3[...], preferred_element_type=jnp.float32) + b3[...], 0.0)
  o = jnp.dot(z, w4[...], preferred_element_type=jnp.float32) + b4[...]
  if has_res:
    o = o + res[...]
  out[...] = o
  if out_bf16:
    out_b[...] = o.astype(jnp.bfloat16)


def _mlp_call(parts, w2, w3, w4, biases, residual=None, block_rows=2000,
              out_bf16=False):
  xs = [p[0] for p in parts]
  w1s = [p[1] for p in parts]
  m = xs[0].shape[0]
  h_out = w4.shape[1]
  grid = m // block_rows
  n_parts = len(parts)
  has_res = residual is not None

  in_specs = []
  for x in xs:
    d = x.shape[1]
    in_specs.append(pl.BlockSpec((block_rows, d), lambda i: (i, 0)))
  for w in w1s + [w2, w3, w4]:
    in_specs.append(
        pl.BlockSpec(w.shape, lambda i: (0, 0)))
  bias2d = [b.reshape(1, -1) for b in biases]
  for b in bias2d:
    in_specs.append(pl.BlockSpec(b.shape, lambda i: (0, 0)))
  args = xs + w1s + [w2, w3, w4] + bias2d
  if has_res:
    in_specs.append(pl.BlockSpec((block_rows, h_out), lambda i: (i, 0)))
    args.append(residual)

  out_spec = pl.BlockSpec((block_rows, h_out), lambda i: (i, 0))
  out_shape = jax.ShapeDtypeStruct((m, h_out), jnp.float32)
  if out_bf16:
    out_specs = (out_spec, out_spec)
    out_shapes = (out_shape, jax.ShapeDtypeStruct((m, h_out), jnp.bfloat16))
  else:
    out_specs = out_spec
    out_shapes = out_shape
  return pl.pallas_call(
      functools.partial(_mlp_body, n_parts, has_res, out_bf16),
      grid=(grid,),
      in_specs=in_specs,
      out_specs=out_specs,
      out_shape=out_shapes,
  )(*args)


def _pool_dec_body(inv_n, *refs):
  (h, w1, w2, w3, w4, b1, b2, b3, b4, out, acc) = refs
  i = pl.program_id(0)

  @pl.when(i == 0)
  def _():
    acc[...] = jnp.zeros_like(acc)

  blk = h[...]
  acc[...] += jnp.sum(blk.reshape(-1, 8, blk.shape[1]), axis=0)

  @pl.when(i == pl.num_programs(0) - 1)
  def _():
    pooled = jnp.sum(acc[...], axis=0, keepdims=True) * inv_n
    z = jnp.maximum(
        jnp.dot(pooled, w1[...], preferred_element_type=jnp.float32) + b1[...],
        0.0)
    z = jnp.maximum(
        jnp.dot(z, w2[...], preferred_element_type=jnp.float32) + b2[...], 0.0)
    z = jnp.maximum(
        jnp.dot(z, w3[...], preferred_element_type=jnp.float32) + b3[...], 0.0)
    out[...] = jnp.dot(z, w4[...], preferred_element_type=jnp.float32) + b4[...]


def _pool_decode(h, dec_params, block_rows=2000):
  n, feat = h.shape
  grid = n // block_rows
  ws = [p["W"] for p in dec_params]
  bs = [p["b"].reshape(1, -1) for p in dec_params]
  in_specs = [pl.BlockSpec((block_rows, feat), lambda i: (i, 0))]
  for w in ws:
    in_specs.append(pl.BlockSpec(w.shape, lambda i: (0, 0)))
  for b in bs:
    in_specs.append(pl.BlockSpec(b.shape, lambda i: (0, 0)))
  out = pl.pallas_call(
      functools.partial(_pool_dec_body, 1.0 / n),
      grid=(grid,),
      in_specs=in_specs,
      out_specs=pl.BlockSpec((1, 1), lambda i: (0, 0)),
      out_shape=jax.ShapeDtypeStruct((1, 1), jnp.float32),
      scratch_shapes=[pltpu.VMEM((8, feat), jnp.float32)],
  )(h, *ws, *bs)
  return out.reshape(())


def _sc_gather(h, src, dst):
  n, feat = h.shape
  e = src.shape[0]
  return _make_gather(n, e, feat, h.dtype)(h, src, dst)


def _pack_bf16(hb):
  n, f = hb.shape
  return jax.lax.bitcast_convert_type(hb.reshape(n, f // 2, 2), jnp.float32)


def _unpack_bf16(g):
  n, f2 = g.shape
  return jax.lax.bitcast_convert_type(g, jnp.bfloat16).reshape(n, f2 * 2)


def _sc_scatter(rows, dst, n_nodes, zero):
  e, feat = rows.shape
  return _make_scatter(n_nodes, e, feat)(rows, dst, zero)


def kernel(x, edge_index, edge_attr, params):
  n, feat = x.shape
  src = edge_index[0]
  dst = edge_index[1]
  zero = jnp.zeros((n, feat), jnp.float32)

  enc_n = params["node_enc"]
  h, h_b = _mlp_call(
      [(x, enc_n[0]["W"])], enc_n[1]["W"], enc_n[2]["W"], enc_n[3]["W"],
      [p["b"] for p in enc_n], out_bf16=True)
  enc_e = params["edge_enc"]
  e = _mlp_call(
      [(edge_attr, enc_e[0]["W"])], enc_e[1]["W"], enc_e[2]["W"],
      enc_e[3]["W"], [p["b"] for p in enc_e])

  for blk in params["blocks"]:
    em = blk["edge_mlp"]
    w1 = em[0]["W"]
    h_src_p, h_dst_p = _sc_gather(_pack_bf16(h_b), src, dst)
    h_src = _unpack_bf16(h_src_p)
    h_dst = _unpack_bf16(h_dst_p)
    e = _mlp_call(
        [(h_src, w1[:feat]), (h_dst, w1[feat:2 * feat]), (e, w1[2 * feat:])],
        em[1]["W"], em[2]["W"], em[3]["W"], [p["b"] for p in em],
        residual=e)
    agg = _sc_scatter(e, dst, n, zero)
    nm = blk["node_mlp"]
    nw1 = nm[0]["W"]
    h, h_b = _mlp_call(
        [(h, nw1[:feat]), (agg[0], nw1[feat:]), (agg[1], nw1[feat:])],
        nm[1]["W"], nm[2]["W"], nm[3]["W"], [p["b"] for p in nm],
        residual=h, out_bf16=True)

  return _pool_decode(h, params["decoder"])


# speedup vs baseline: 3.4225x; 3.4225x over previous
import functools

import jax
import jax.numpy as jnp
from jax import lax
from jax.experimental import pallas as pl
from jax.experimental.pallas import tpu as pltpu
from jax.experimental.pallas import tpu_sc as plsc

_NC = 2
_NS = 16
_NW = _NC * _NS
_CH = 128
_NB = 4


def _sc_mesh():
  return plsc.VectorSubcoreMesh(
      core_axis_name="c", subcore_axis_name="s", num_cores=_NC,
      num_subcores=_NS)


@functools.lru_cache(maxsize=None)
def _make_gather(n_nodes, n_edges, feat):
  n_chunks = n_edges // _CH
  per_w = (-(-n_chunks // _NW) + 7) // 8 * 8
  groups = -(-per_w // _NB)

  @functools.partial(
      pl.kernel,
      out_type=(
          jax.ShapeDtypeStruct((n_edges, feat), jnp.float32),
          jax.ShapeDtypeStruct((n_edges, feat), jnp.float32),
      ),
      mesh=_sc_mesh(),
      scratch_types=[
          pltpu.VMEM((per_w, _CH), jnp.int32),
          pltpu.VMEM((_NB, _CH, feat), jnp.float32),
          [pltpu.SemaphoreType.DMA] * _NB,
          [pltpu.SemaphoreType.DMA] * _NB,
      ],
  )
  def gather_kernel(h_hbm, src_hbm, dst_hbm, osrc_hbm, odst_hbm,
                    idx_all, rows, sg, ss):
    wid = lax.axis_index("s") * _NC + lax.axis_index("c")
    lo = wid * per_w
    n_my = jnp.minimum(per_w, n_chunks - lo)

    for idx_hbm, out_hbm in ((src_hbm, osrc_hbm), (dst_hbm, odst_hbm)):
      pltpu.sync_copy(idx_hbm.at[pl.ds(lo, per_w)], idx_all)

      def start_gather(c, b):
        pltpu.async_copy(h_hbm.at[idx_all.at[c]], rows.at[b], sg[b])

      for b in range(_NB):
        @pl.when(b < n_my)
        def _(b=b):
          start_gather(b, b)

      @pl.loop(0, groups)
      def _(g, out_hbm=out_hbm):
        for b in range(_NB):
          c = g * _NB + b

          @pl.when(c < n_my)
          def _(b=b, c=c, out_hbm=out_hbm):
            pltpu.make_async_copy(
                h_hbm.at[idx_all.at[c]], rows.at[b], sg[b]).wait()
            dst_slice = out_hbm.at[pl.ds((lo + c) * _CH, _CH)]
            pltpu.async_copy(rows.at[b], dst_slice, ss[b])
            nc = c + _NB

            @pl.when(nc < n_my)
            def _():
              pltpu.make_async_copy(rows.at[b], dst_slice, ss[b]).wait()
              start_gather(nc, b)

      for b in range(_NB):
        @pl.when(b < n_my)
        def _(b=b, out_hbm=out_hbm):
          pltpu.make_async_copy(
              rows.at[b], out_hbm.at[pl.ds(0, _CH)], ss[b]).wait()

  return gather_kernel


@functools.lru_cache(maxsize=None)
def _make_scatter(n_nodes, n_edges, feat):
  n_chunks = n_edges // _CH
  per_w = (-(-n_chunks // _NW) + 7) // 8 * 8
  nb = 2
  groups = -(-per_w // nb)
  out_ch = 80
  n_out_chunks = -(-n_nodes // out_ch)
  out_per_sub = -(-n_out_chunks // _NS)

  @functools.partial(
      pl.kernel,
      out_type=jax.ShapeDtypeStruct((_NC, n_nodes, feat), jnp.float32),
      mesh=_sc_mesh(),
      scratch_types=[
          pltpu.VMEM((per_w, _CH), jnp.int32),
          pltpu.VMEM((nb, _CH, feat), jnp.float32),
          pltpu.VMEM_SHARED((n_nodes, feat), jnp.float32),
          [pltpu.SemaphoreType.DMA] * nb,
          [pltpu.SemaphoreType.DMA] * nb,
      ],
  )
  def scatter_kernel(rows_hbm, dst_hbm, zero_hbm, out_hbm,
                     idx_all, rows, acc_sh, sl, sa):
    cid = lax.axis_index("c")
    sid = lax.axis_index("s")
    wid = sid * _NC + cid
    lo = wid * per_w
    n_my = jnp.minimum(per_w, n_chunks - lo)

    @pl.loop(0, out_per_sub)
    def _(c):
      ch = sid + _NS * c

      @pl.when(ch < n_out_chunks)
      def _():
        base = ch * out_ch
        pltpu.sync_copy(zero_hbm.at[pl.ds(base, out_ch)],
                        acc_sh.at[pl.ds(base, out_ch)])

    pltpu.sync_copy(dst_hbm.at[pl.ds(lo, per_w)], idx_all)
    plsc.subcore_barrier()

    def start_load(c, b):
      pltpu.async_copy(
          rows_hbm.at[pl.ds((lo + c) * _CH, _CH)], rows.at[b], sl[b])

    for b in range(nb):
      @pl.when(b < n_my)
      def _(b=b):
        start_load(b, b)

    @pl.loop(0, groups)
    def _(g):
      for b in range(nb):
        c = g * nb + b

        @pl.when(c < n_my)
        def _(b=b, c=c):
          pltpu.make_async_copy(
              rows_hbm.at[pl.ds(lo * _CH, _CH)], rows.at[b], sl[b]).wait()
          pltpu.async_copy(rows.at[b], acc_sh.at[idx_all.at[c]], sa[b],
                           add=True)
          nc = c + nb

          @pl.when(nc < n_my)
          def _():
            pltpu.make_async_copy(
                rows.at[b], acc_sh.at[idx_all.at[c]], sa[b]).wait()
            start_load(nc, b)

    for b in range(nb):
      @pl.when(b < n_my)
      def _(b=b):
        pltpu.make_async_copy(
            rows.at[b], acc_sh.at[idx_all.at[0]], sa[b]).wait()

    plsc.subcore_barrier()

    @pl.loop(0, out_per_sub)
    def _(c):
      ch = sid + _NS * c

      @pl.when(ch < n_out_chunks)
      def _():
        base = ch * out_ch
        pltpu.sync_copy(acc_sh.at[pl.ds(base, out_ch)],
                        out_hbm.at[cid, pl.ds(base, out_ch)])

  return scatter_kernel


def _mlp_body(n_parts, has_res, out_bf16, *refs):
  xs = refs[:n_parts]
  w1s = refs[n_parts:2 * n_parts]
  w2, w3, w4 = refs[2 * n_parts:2 * n_parts + 3]
  b1, b2, b3, b4 = refs[2 * n_parts + 3:2 * n_parts + 7]
  pos = 2 * n_parts + 7
  res = refs[pos] if has_res else None
  pos += 1 if has_res else 0
  out = refs[pos]
  out_b = refs[pos + 1] if out_bf16 else None

  def dot1(x, w):
    return jnp.dot(x, w.astype(x.dtype), preferred_element_type=jnp.float32)

  z = dot1(xs[0][...], w1s[0][...])
  for j in range(1, n_parts):
    z += dot1(xs[j][...], w1s[j][...])
  z = jnp.maximum(z + b1[...], 0.0)
  z = jnp.maximum(
      jnp.dot(z, w2[...], preferred_element_type=jnp.float32) + b2[...], 0.0)
  z = jnp.maximum(
      jnp.dot(z, w3[...], preferred_element_type=jnp.float32) + b3[...], 0.0)
  o = jnp.dot(z, w4[...], preferred_element_type=jnp.float32) + b4[...]
  if has_res:
    o = o + res[...]
  out[...] = o
  if out_bf16:
    out_b[...] = o.astype(jnp.bfloat16)


def _mlp_call(parts, w2, w3, w4, biases, residual=None, block_rows=2000,
              out_bf16=False):
  xs = [p[0] for p in parts]
  w1s = [p[1] for p in parts]
  m = xs[0].shape[0]
  h_out = w4.shape[1]
  grid = m // block_rows
  n_parts = len(parts)
  has_res = residual is not None

  in_specs = []
  for x in xs:
    d = x.shape[1]
    in_specs.append(pl.BlockSpec((block_rows, d), lambda i: (i, 0)))
  for w in w1s + [w2, w3, w4]:
    in_specs.append(
        pl.BlockSpec(w.shape, lambda i: (0, 0)))
  bias2d = [b.reshape(1, -1) for b in biases]
  for b in bias2d:
    in_specs.append(pl.BlockSpec(b.shape, lambda i: (0, 0)))
  args = xs + w1s + [w2, w3, w4] + bias2d
  if has_res:
    in_specs.append(pl.BlockSpec((block_rows, h_out), lambda i: (i, 0)))
    args.append(residual)

  out_spec = pl.BlockSpec((block_rows, h_out), lambda i: (i, 0))
  out_shape = jax.ShapeDtypeStruct((m, h_out), jnp.float32)
  if out_bf16:
    out_specs = (out_spec, out_spec)
    out_shapes = (out_shape, jax.ShapeDtypeStruct((m, h_out), jnp.bfloat16))
  else:
    out_specs = out_spec
    out_shapes = out_shape
  return pl.pallas_call(
      functools.partial(_mlp_body, n_parts, has_res, out_bf16),
      grid=(grid,),
      in_specs=in_specs,
      out_specs=out_specs,
      out_shape=out_shapes,
  )(*args)


def _pool_dec_body(inv_n, *refs):
  (h, w1, w2, w3, w4, b1, b2, b3, b4, out, acc) = refs
  i = pl.program_id(0)

  @pl.when(i == 0)
  def _():
    acc[...] = jnp.zeros_like(acc)

  blk = h[...]
  acc[...] += jnp.sum(blk.reshape(-1, 8, blk.shape[1]), axis=0)

  @pl.when(i == pl.num_programs(0) - 1)
  def _():
    pooled = jnp.sum(acc[...], axis=0, keepdims=True) * inv_n
    z = jnp.maximum(
        jnp.dot(pooled, w1[...], preferred_element_type=jnp.float32) + b1[...],
        0.0)
    z = jnp.maximum(
        jnp.dot(z, w2[...], preferred_element_type=jnp.float32) + b2[...], 0.0)
    z = jnp.maximum(
        jnp.dot(z, w3[...], preferred_element_type=jnp.float32) + b3[...], 0.0)
    out[...] = jnp.dot(z, w4[...], preferred_element_type=jnp.float32) + b4[...]


def _pool_decode(h, dec_params, block_rows=2000):
  n, feat = h.shape
  grid = n // block_rows
  ws = [p["W"] for p in dec_params]
  bs = [p["b"].reshape(1, -1) for p in dec_params]
  in_specs = [pl.BlockSpec((block_rows, feat), lambda i: (i, 0))]
  for w in ws:
    in_specs.append(pl.BlockSpec(w.shape, lambda i: (0, 0)))
  for b in bs:
    in_specs.append(pl.BlockSpec(b.shape, lambda i: (0, 0)))
  out = pl.pallas_call(
      functools.partial(_pool_dec_body, 1.0 / n),
      grid=(grid,),
      in_specs=in_specs,
      out_specs=pl.BlockSpec((1, 1), lambda i: (0, 0)),
      out_shape=jax.ShapeDtypeStruct((1, 1), jnp.float32),
      scratch_shapes=[pltpu.VMEM((8, feat), jnp.float32)],
  )(h, *ws, *bs)
  return out.reshape(())


def _pad_idx(idx, n_edges):
  n_chunks = n_edges // _CH
  per_w = (-(-n_chunks // _NW) + 7) // 8 * 8
  total = _NW * per_w * _CH
  return jnp.pad(idx, (0, total - n_edges)).reshape(-1, _CH)


def _sc_gather(h, src2, dst2, n_edges):
  n, feat = h.shape
  return _make_gather(n, n_edges, feat)(h, src2, dst2)


def _sc_scatter(rows, dst2, n_nodes, zero):
  e, feat = rows.shape
  return _make_scatter(n_nodes, e, feat)(rows, dst2, zero)


def kernel(x, edge_index, edge_attr, params):
  n, feat = x.shape
  n_edges = edge_index.shape[1]
  src2 = _pad_idx(edge_index[0], n_edges)
  dst2 = _pad_idx(edge_index[1], n_edges)
  zero = jnp.zeros((n, feat), jnp.float32)

  enc_n = params["node_enc"]
  h = _mlp_call(
      [(x, enc_n[0]["W"])], enc_n[1]["W"], enc_n[2]["W"], enc_n[3]["W"],
      [p["b"] for p in enc_n])
  enc_e = params["edge_enc"]
  e = _mlp_call(
      [(edge_attr, enc_e[0]["W"])], enc_e[1]["W"], enc_e[2]["W"],
      enc_e[3]["W"], [p["b"] for p in enc_e])

  for blk in params["blocks"]:
    em = blk["edge_mlp"]
    w1 = em[0]["W"]
    h_src, h_dst = _sc_gather(h, src2, dst2, n_edges)
    e = _mlp_call(
        [(h_src, w1[:feat]), (h_dst, w1[feat:2 * feat]), (e, w1[2 * feat:])],
        em[1]["W"], em[2]["W"], em[3]["W"], [p["b"] for p in em],
        residual=e)
    agg = _sc_scatter(e, dst2, n, zero)
    nm = blk["node_mlp"]
    nw1 = nm[0]["W"]
    h = _mlp_call(
        [(h, nw1[:feat]), (agg[0], nw1[feat:]), (agg[1], nw1[feat:])],
        nm[1]["W"], nm[2]["W"], nm[3]["W"], [p["b"] for p in nm],
        residual=h)

  return _pool_decode(h, params["decoder"])
